# baseline (speedup 1.0000x reference)
import functools

import jax
import jax.numpy as jnp
from jax import lax
from jax.experimental import pallas as pl
from jax.experimental.pallas import tpu as pltpu
from jax.experimental.pallas import tpu_sc as plsc

VOCAB = 1000000
DIM = 32
BATCH = 4096
SEQ = 200

BBLK = 128
NBUF = 4
NTR = 2


@jax.jit
def _embed(table, xt):
    info = plsc.get_sparse_core_info()
    nc, nl = info.num_cores, info.num_lanes

    mesh = plsc.VectorSubcoreMesh(core_axis_name="c", subcore_axis_name="s")

    @functools.partial(
        pl.kernel,
        mesh=mesh,
        out_type=jax.ShapeDtypeStruct((SEQ, DIM, BATCH), jnp.float32),
        scratch_types=[
            pltpu.VMEM((SEQ, BBLK), jnp.int32),
            pltpu.VMEM((NBUF, BBLK, DIM), jnp.float32),
            pltpu.VMEM((NTR, DIM, BBLK + 1), jnp.float32),
            pltpu.SemaphoreType.DMA((NBUF,)),
            pltpu.SemaphoreType.DMA((NTR,)),
        ],
        compiler_params=pltpu.CompilerParams(
            use_tc_tiling_on_sc=False, needs_layout_passes=False),
    )
    def k(xt_hbm, tbl, out_hbm, idx_v, rows_v, tr_v, gsem, wsem):
        wid = lax.axis_index("s") * nc + lax.axis_index("c")
        bbase = wid * BBLK
        pltpu.sync_copy(xt_hbm.at[:, pl.ds(bbase, BBLK)], idx_v)

        lane_h = [lax.iota(jnp.int32, nl) + h * nl for h in range(DIM // nl)]

        def fire(g, rb):
            pltpu.async_copy(tbl.at[idx_v.at[g]], rows_v.at[rb], gsem.at[rb])

        def drain_gather(rb):
            pltpu.make_async_copy(
                tbl.at[pl.ds(0, BBLK)], rows_v.at[rb], gsem.at[rb]).wait()

        def transpose(rb, tb):
            def ibody(i0, _):
                for u in range(16):
                    i = i0 * 16 + u
                    col = jnp.full((nl,), i, jnp.int32)
                    for h in range(DIM // nl):
                        vals = rows_v[rb, i, pl.ds(h * nl, nl)]
                        plsc.store_scatter(
                            tr_v.at[tb], [lane_h[h], col], vals)
                return 0
            lax.fori_loop(0, BBLK // 16, ibody, 0)

        def write(g, tb):
            pltpu.async_copy(
                tr_v.at[tb, :, pl.ds(0, BBLK)],
                out_hbm.at[g, :, pl.ds(bbase, BBLK)],
                wsem.at[tb])

        def wait_write(tb):
            pltpu.make_async_copy(
                tr_v.at[tb, :, pl.ds(0, BBLK)],
                out_hbm.at[0, :, pl.ds(0, BBLK)], wsem.at[tb]
            ).wait()

        fire(0, 0)
        fire(1, 1)
        fire(2, 2)
        drain_gather(0)
        transpose(0, 0)
        write(0, 0)
        fire(3, 3)
        drain_gather(1)
        transpose(1, 1)
        write(1, 1)

        def body(i, _):
            for b in range(NBUF):
                g = 2 + i * NBUF + b
                rb = (2 + b) % NBUF
                tb = b % NTR
                wait_write(tb)
                fire(g + 2, b)
                drain_gather(rb)
                transpose(rb, tb)
                write(g, tb)
            return 0

        lax.fori_loop(0, (SEQ - 4) // NBUF, body, 0)

        for g in (SEQ - 2, SEQ - 1):
            rb = g % NBUF
            tb = g % NTR
            wait_write(tb)
            drain_gather(rb)
            transpose(rb, tb)
            write(g, tb)
        wait_write(0)
        wait_write(1)

    return k(xt, table)


VB = 4096


def _linearize_tc(tt):
    def body(i_ref, o_ref):
        xt = i_ref[...].T.reshape(VB // 4, 4, DIM)
        for m in range(4):
            o_ref[:, m * DIM:(m + 1) * DIM] = xt[:, m, :]
    return pl.pallas_call(
        body,
        grid=(pl.cdiv(VOCAB, VB),),
        in_specs=[pl.BlockSpec((DIM, VB), lambda i: (0, i))],
        out_specs=pl.BlockSpec((VB // 4, 128), lambda i: (i, 0)),
        out_shape=jax.ShapeDtypeStruct((VOCAB // 4, 128), jnp.float32),
    )(tt)


def kernel(x, table):
    xt = x.T
    t128 = _linearize_tc(table.T)
    tlin = t128.reshape(VOCAB, DIM)
    out_t = _embed(tlin, xt)
    return out_t.transpose(2, 0, 1)


# speedup vs baseline: 1.0620x; 1.0620x over previous
import functools

import jax
import jax.numpy as jnp
from jax import lax
from jax.experimental import pallas as pl
from jax.experimental.pallas import tpu as pltpu
from jax.experimental.pallas import tpu_sc as plsc

VOCAB = 1000000
DIM = 32
BATCH = 4096
SEQ = 200

BBLK = 128
NBUF = 4
NTR = 2


@jax.jit
def _embed(table, xt):
    info = plsc.get_sparse_core_info()
    nc, nl = info.num_cores, info.num_lanes

    mesh = plsc.VectorSubcoreMesh(core_axis_name="c", subcore_axis_name="s")

    @functools.partial(
        pl.kernel,
        mesh=mesh,
        out_type=jax.ShapeDtypeStruct((SEQ, DIM, BATCH), jnp.float32),
        scratch_types=[
            pltpu.VMEM((SEQ, BBLK), jnp.int32),
            pltpu.VMEM((NBUF, BBLK, DIM), jnp.float32),
            pltpu.VMEM((NTR, DIM, BBLK + 1), jnp.float32),
            pltpu.SemaphoreType.DMA((NBUF,)),
            pltpu.SemaphoreType.DMA((NTR,)),
        ],
        compiler_params=pltpu.CompilerParams(
            use_tc_tiling_on_sc=False, needs_layout_passes=False),
    )
    def k(xt_hbm, tbl, out_hbm, idx_v, rows_v, tr_v, gsem, wsem):
        wid = lax.axis_index("s") * nc + lax.axis_index("c")
        bbase = wid * BBLK
        pltpu.sync_copy(xt_hbm.at[:, pl.ds(bbase, BBLK)], idx_v)

        lane_h = [lax.iota(jnp.int32, nl) + h * nl for h in range(DIM // nl)]

        def fire(g, rb):
            pltpu.async_copy(tbl.at[idx_v.at[g]], rows_v.at[rb], gsem.at[rb])

        def drain_gather(rb):
            pltpu.make_async_copy(
                tbl.at[pl.ds(0, BBLK)], rows_v.at[rb], gsem.at[rb]).wait()

        def transpose(rb, tb):
            def ibody(i0, _):
                for u in range(8):
                    i = i0 * 8 + u
                    col = jnp.full((nl,), i, jnp.int32)
                    for h in range(DIM // nl):
                        vals = rows_v[rb, i, pl.ds(h * nl, nl)]
                        plsc.store_scatter(
                            tr_v.at[tb], [lane_h[h], col], vals)
                return 0
            lax.fori_loop(0, BBLK // 8, ibody, 0)

        def write(g, tb):
            pltpu.async_copy(
                tr_v.at[tb, :, pl.ds(0, BBLK)],
                out_hbm.at[g, :, pl.ds(bbase, BBLK)],
                wsem.at[tb])

        def wait_write(tb):
            pltpu.make_async_copy(
                tr_v.at[tb, :, pl.ds(0, BBLK)],
                out_hbm.at[0, :, pl.ds(0, BBLK)], wsem.at[tb]
            ).wait()

        fire(0, 0)
        fire(1, 1)
        fire(2, 2)
        drain_gather(0)
        transpose(0, 0)
        write(0, 0)
        fire(3, 3)
        drain_gather(1)
        transpose(1, 1)
        write(1, 1)

        def body(i, _):
            for b in range(NBUF):
                g = 2 + i * NBUF + b
                rb = (2 + b) % NBUF
                tb = b % NTR
                wait_write(tb)
                fire(g + 2, b)
                drain_gather(rb)
                transpose(rb, tb)
                write(g, tb)
            return 0

        lax.fori_loop(0, (SEQ - 4) // NBUF, body, 0)

        for g in (SEQ - 2, SEQ - 1):
            rb = g % NBUF
            tb = g % NTR
            wait_write(tb)
            drain_gather(rb)
            transpose(rb, tb)
            write(g, tb)
        wait_write(0)
        wait_write(1)

    return k(xt, table)


VB = 8192


def _linearize_tc(tt):
    def body(i_ref, o_ref):
        xt = i_ref[...].T.reshape(VB // 4, 4, DIM)
        for m in range(4):
            o_ref[:, m * DIM:(m + 1) * DIM] = xt[:, m, :]
    return pl.pallas_call(
        body,
        grid=(pl.cdiv(VOCAB, VB),),
        in_specs=[pl.BlockSpec((DIM, VB), lambda i: (0, i))],
        out_specs=pl.BlockSpec((VB // 4, 128), lambda i: (i, 0)),
        out_shape=jax.ShapeDtypeStruct((VOCAB // 4, 128), jnp.float32),
    )(tt)


def kernel(x, table):
    xt = x.T
    t128 = _linearize_tc(table.T)
    tlin = t128.reshape(VOCAB, DIM)
    out_t = _embed(tlin, xt)
    return out_t.transpose(2, 0, 1)


# speedup vs baseline: 1.0801x; 1.0170x over previous
import functools

import jax
import jax.numpy as jnp
from jax import lax
from jax.experimental import pallas as pl
from jax.experimental.pallas import tpu as pltpu
from jax.experimental.pallas import tpu_sc as plsc

VOCAB = 1000000
DIM = 32
BATCH = 4096
SEQ = 200

BBLK = 128
NBUF = 4
NTR = 2


@jax.jit
def _embed(table, xt):
    info = plsc.get_sparse_core_info()
    nc, nl = info.num_cores, info.num_lanes

    mesh = plsc.VectorSubcoreMesh(core_axis_name="c", subcore_axis_name="s")

    @functools.partial(
        pl.kernel,
        mesh=mesh,
        out_type=jax.ShapeDtypeStruct((SEQ, DIM, BATCH), jnp.float32),
        scratch_types=[
            pltpu.VMEM((SEQ, BBLK), jnp.int32),
            pltpu.VMEM((NBUF, BBLK, DIM), jnp.float32),
            pltpu.VMEM((NTR, DIM, BBLK + 1), jnp.float32),
            pltpu.SemaphoreType.DMA((NBUF,)),
            pltpu.SemaphoreType.DMA((NTR,)),
        ],
        compiler_params=pltpu.CompilerParams(
            use_tc_tiling_on_sc=False, needs_layout_passes=False),
    )
    def k(xt_hbm, tbl, out_hbm, idx_v, rows_v, tr_v, gsem, wsem):
        wid = lax.axis_index("s") * nc + lax.axis_index("c")
        bbase = wid * BBLK
        pltpu.sync_copy(xt_hbm.at[:, pl.ds(bbase, BBLK)], idx_v)

        lane_h = [lax.iota(jnp.int32, nl) + h * nl for h in range(DIM // nl)]

        def fire(g, rb):
            pltpu.async_copy(tbl.at[idx_v.at[g]], rows_v.at[rb], gsem.at[rb])

        def drain_gather(rb):
            pltpu.make_async_copy(
                tbl.at[pl.ds(0, BBLK)], rows_v.at[rb], gsem.at[rb]).wait()

        def transpose(rb, tb):
            def ibody(i0, _):
                for u in range(8):
                    i = i0 * 8 + u
                    col = jnp.full((nl,), i, jnp.int32)
                    for h in range(DIM // nl):
                        vals = rows_v[rb, i, pl.ds(h * nl, nl)]
                        plsc.store_scatter(
                            tr_v.at[tb], [lane_h[h], col], vals)
                return 0
            lax.fori_loop(0, BBLK // 8, ibody, 0)

        def write(g, tb):
            pltpu.async_copy(
                tr_v.at[tb, :, pl.ds(0, BBLK)],
                out_hbm.at[g, :, pl.ds(bbase, BBLK)],
                wsem.at[tb])

        def wait_write(tb):
            pltpu.make_async_copy(
                tr_v.at[tb, :, pl.ds(0, BBLK)],
                out_hbm.at[0, :, pl.ds(0, BBLK)], wsem.at[tb]
            ).wait()

        fire(0, 0)
        fire(1, 1)
        fire(2, 2)
        drain_gather(0)
        transpose(0, 0)
        write(0, 0)
        fire(3, 3)
        drain_gather(1)
        transpose(1, 1)
        write(1, 1)

        def body(i, _):
            for b in range(NBUF):
                g = 2 + i * NBUF + b
                rb = (2 + b) % NBUF
                tb = b % NTR
                wait_write(tb)
                fire(g + 2, b)
                drain_gather(rb)
                transpose(rb, tb)
                write(g, tb)
            return 0

        lax.fori_loop(0, (SEQ - 4) // NBUF, body, 0)

        for g in (SEQ - 2, SEQ - 1):
            rb = g % NBUF
            tb = g % NTR
            wait_write(tb)
            drain_gather(rb)
            transpose(rb, tb)
            write(g, tb)
        wait_write(0)
        wait_write(1)

    return k(xt, table)


VB = 16384


def _linearize_tc(tt):
    def body(i_ref, o_ref):
        xt = i_ref[...].T.reshape(VB // 4, 4, DIM)
        for m in range(4):
            o_ref[:, m * DIM:(m + 1) * DIM] = xt[:, m, :]
    return pl.pallas_call(
        body,
        grid=(pl.cdiv(VOCAB, VB),),
        in_specs=[pl.BlockSpec((DIM, VB), lambda i: (0, i))],
        out_specs=pl.BlockSpec((VB // 4, 128), lambda i: (i, 0)),
        out_shape=jax.ShapeDtypeStruct((VOCAB // 4, 128), jnp.float32),
    )(tt)


def kernel(x, table):
    xt = x.T
    t128 = _linearize_tc(table.T)
    tlin = t128.reshape(VOCAB, DIM)
    out_t = _embed(tlin, xt)
    return out_t.transpose(2, 0, 1)
